# Initial kernel scaffold; baseline (speedup 1.0000x reference)
import jax
import jax.numpy as jnp
from jax.experimental import pallas as pl


def kernel(feat0, feat1, feat2, loc_t, conf_t):
    raise NotImplementedError("write your pallas kernel here")


# speedup vs baseline: 3.0998x; 3.0998x over previous
import functools

import jax
import jax.numpy as jnp
from jax import lax
from jax.experimental import pallas as pl
from jax.experimental.pallas import tpu as pltpu
from jax.experimental.pallas import tpu_sc as plsc

_NUM_CLASSES = 10
_NUM_ATTR = _NUM_CLASSES + 4
_NUM_PRIORS = 64 * 64 + 32 * 32 + 16 * 16
_NEGPOS_RATIO = 3
_BATCH = 32
_L = 16
_NV = _NUM_PRIORS // _L


def _dense_body(p_ref, loc_ref, conf_ref, lossneg_ref, part_ref):
    x = p_ref[0]
    conf = x[4:, :]
    m = jnp.max(conf, axis=0, keepdims=True)
    s = jnp.sum(jnp.exp(conf - m), axis=0, keepdims=True)
    lse = m + jnp.log(s)
    c = conf_ref[0]
    gathered = jnp.zeros_like(lse)
    for k in range(_NUM_CLASSES):
        gathered = jnp.where(c == k, conf[k:k + 1, :], gathered)
    loss_c_all = lse - gathered
    pos = c > 0
    posf = pos.astype(jnp.float32)
    lossneg_ref[0] = jnp.where(pos, 0.0, loss_c_all)

    pos_cnt = jnp.sum(posf)
    pos_sum = jnp.sum(loss_c_all * posf)
    d = x[:4, :] - loc_ref[0]
    ad = jnp.abs(d)
    sl1 = jnp.where(ad < 1.0, 0.5 * d * d, ad - 0.5)
    loc_sum = jnp.sum(sl1 * posf)

    lane = lax.broadcasted_iota(jnp.int32, (1, 128), 1)
    row = jnp.where(lane == 0, pos_cnt, 0.0)
    row = jnp.where(lane == 1, pos_sum, row)
    row = jnp.where(lane == 2, loc_sum, row)
    part_ref[0] = row


def _dense(p, loc_t_t, conf3):
    return pl.pallas_call(
        _dense_body,
        grid=(_BATCH,),
        in_specs=[
            pl.BlockSpec((1, _NUM_ATTR, _NUM_PRIORS), lambda b: (b, 0, 0)),
            pl.BlockSpec((1, 4, _NUM_PRIORS), lambda b: (b, 0, 0)),
            pl.BlockSpec((1, 1, _NUM_PRIORS), lambda b: (b, 0, 0)),
        ],
        out_specs=[
            pl.BlockSpec((1, 1, _NUM_PRIORS), lambda b: (b, 0, 0)),
            pl.BlockSpec((1, 1, 128), lambda b: (b, 0, 0)),
        ],
        out_shape=[
            jax.ShapeDtypeStruct((_BATCH, 1, _NUM_PRIORS), jnp.float32),
            jax.ShapeDtypeStruct((_BATCH, 1, 128), jnp.float32),
        ],
    )(p, loc_t_t, conf3)


def _sc_topk(lossneg, partials):
    mesh = plsc.VectorSubcoreMesh(core_axis_name="c", subcore_axis_name="s")

    @functools.partial(
        pl.kernel,
        mesh=mesh,
        out_type=jax.ShapeDtypeStruct((_BATCH, _L), jnp.float32),
        scratch_types=[
            pltpu.VMEM((_NUM_PRIORS,), jnp.float32),
            pltpu.VMEM((_BATCH, _L), jnp.float32),
            pltpu.VMEM((_L,), jnp.float32),
        ],
        compiler_params=pltpu.CompilerParams(needs_layout_passes=False),
    )
    def body(loss_hbm, part_hbm, out_hbm, loss_v, part_v, res_v):
        cid = lax.axis_index("c")
        sid = lax.axis_index("s")
        w = sid * 2 + cid
        pltpu.sync_copy(loss_hbm.at[w], loss_v)
        pltpu.sync_copy(part_hbm, part_v)

        lane = lax.iota(jnp.int32, 16)
        acc = jnp.zeros((_L,), jnp.float32)
        mine = jnp.zeros((_L,), jnp.float32)
        for i in range(_BATCH):
            row = part_v[i, :]
            acc = acc + row
            mine = jnp.where(w == i, row, mine)
        num_pos_f = jnp.sum(jnp.where(lane == 0, acc, 0.0))
        num_pos = num_pos_f.astype(jnp.int32)
        num_neg = jnp.minimum(_NEGPOS_RATIO * num_pos, _NUM_PRIORS - num_pos)
        own = jnp.sum(jnp.where((lane == 1) | (lane == 2), mine, 0.0))

        def count_ge(cand):
            def inner(j, cacc):
                v = loss_v[pl.ds(j * _L, _L)]
                b = lax.bitcast_convert_type(v, jnp.int32)
                return cacc + jnp.where(b >= cand, 1, 0)
            cvec = lax.fori_loop(0, _NV, inner, jnp.zeros((_L,), jnp.int32))
            return jnp.sum(cvec)

        def bit_step(i, t):
            cand = t | (jnp.int32(1) << (jnp.int32(30) - i))
            cnt = count_ge(cand)
            return jnp.where(cnt >= num_neg, cand, t)

        t_key = lax.fori_loop(0, 31, bit_step, jnp.int32(0))

        def final(j, carry):
            sacc, cacc = carry
            v = loss_v[pl.ds(j * _L, _L)]
            b = lax.bitcast_convert_type(v, jnp.int32)
            gt = b > t_key
            return (sacc + jnp.where(gt, v, 0.0),
                    cacc + jnp.where(gt, 1, 0))

        sacc, cacc = lax.fori_loop(
            0, _NV, final,
            (jnp.zeros((_L,), jnp.float32), jnp.zeros((_L,), jnp.int32)))
        sum_gt = jnp.sum(sacc)
        cnt_gt = jnp.sum(cacc)
        t_vec = lax.bitcast_convert_type(
            jnp.full((_L,), t_key, jnp.int32), jnp.float32)
        t_val = jnp.sum(jnp.where(lane == 0, t_vec, 0.0))
        neg_sum = sum_gt + (num_neg - cnt_gt).astype(jnp.float32) * t_val
        neg_sum = jnp.where(num_neg > 0, neg_sum, 0.0)

        total_w = own + neg_sum
        res_v[...] = jnp.where(lane == 0, total_w,
                               jnp.where(lane == 1, num_pos_f, 0.0))
        pltpu.sync_copy(res_v, out_hbm.at[w])

    return body(lossneg, partials)


def kernel(feat0, feat1, feat2, loc_t, conf_t):
    b = feat0.shape[0]
    p = jnp.concatenate(
        [feat0.reshape(b, _NUM_ATTR, -1),
         feat1.reshape(b, _NUM_ATTR, -1),
         feat2.reshape(b, _NUM_ATTR, -1)], axis=2)
    loc_t_t = jnp.transpose(loc_t, (0, 2, 1))
    conf3 = conf_t.reshape(b, 1, _NUM_PRIORS).astype(jnp.int32)

    lossneg, partials = _dense(p, loc_t_t, conf3)
    out = _sc_topk(lossneg.reshape(b, _NUM_PRIORS),
                   partials[:, 0, :_L])
    num_pos_f = out[0, 1]
    denom = jnp.maximum(num_pos_f, 1.0)
    return jnp.sum(out[:, 0]) / denom


# speedup vs baseline: 3.3457x; 1.0793x over previous
import functools

import jax
import jax.numpy as jnp
from jax import lax
from jax.experimental import pallas as pl
from jax.experimental.pallas import tpu as pltpu
from jax.experimental.pallas import tpu_sc as plsc

_NUM_CLASSES = 10
_NUM_ATTR = _NUM_CLASSES + 4
_NUM_PRIORS = 64 * 64 + 32 * 32 + 16 * 16
_NEGPOS_RATIO = 3
_BATCH = 32
_L = 16
_NV = _NUM_PRIORS // _L
_ROWS = _NUM_PRIORS // 128
_CHUNKS = (32, 8, 2)


def _dense_body(f0_ref, f1_ref, f2_ref, conf_ref, loc_ref,
                lossneg_ref, part_ref):
    b = pl.program_id(0)
    c_all = conf_ref[0]
    l_all = loc_ref[0]

    pos_cnt = jnp.float32(0.0)
    pos_sum = jnp.float32(0.0)
    loc_sum = jnp.float32(0.0)
    off = 0
    for x_ref, h in zip((f0_ref, f1_ref, f2_ref), _CHUNKS):
        x = x_ref[0]
        cc = c_all[off:off + h]
        lc = l_all[:, off:off + h, :]
        conf = x[4:]
        m = jnp.max(conf, axis=0)
        s = jnp.sum(jnp.exp(conf - m[None]), axis=0)
        lse = m + jnp.log(s)
        katt = lax.broadcasted_iota(jnp.int32, (_NUM_CLASSES, h, 128), 0)
        gathered = jnp.max(
            jnp.where(katt == cc[None], conf, -jnp.inf), axis=0)
        loss_c_all = lse - gathered
        pos = cc > 0
        posf = pos.astype(jnp.float32)
        lossneg_ref[0, off:off + h, :] = jnp.where(pos, 0.0, loss_c_all)
        pos_cnt += jnp.sum(posf)
        pos_sum += jnp.sum(loss_c_all * posf)
        d = x[:4] - lc
        ad = jnp.abs(d)
        sl1 = jnp.where(ad < 1.0, 0.5 * d * d, ad - 0.5)
        loc_sum += jnp.sum(sl1 * posf[None])
        off += h

    sub = lax.broadcasted_iota(jnp.int32, (8, 128), 0)
    lane = lax.broadcasted_iota(jnp.int32, (8, 128), 1)
    vals = jnp.where(lane == 4 * b, pos_cnt,
                     jnp.where(lane == 4 * b + 1, pos_sum,
                               jnp.where(lane == 4 * b + 2, loc_sum, 0.0)))
    vals = jnp.where(sub == 0, vals, 0.0)
    mask = (sub == 0) & (lane >= 4 * b) & (lane <= 4 * b + 2)

    @pl.when(b == 0)
    def _():
        part_ref[...] = vals

    @pl.when(b > 0)
    def _():
        part_ref[...] = jnp.where(mask, vals, part_ref[...])


def _dense(f0, f1, f2, conf, loc_t4):
    return pl.pallas_call(
        _dense_body,
        grid=(_BATCH,),
        in_specs=[
            pl.BlockSpec((1, _NUM_ATTR, _CHUNKS[0], 128), lambda b: (b, 0, 0, 0)),
            pl.BlockSpec((1, _NUM_ATTR, _CHUNKS[1], 128), lambda b: (b, 0, 0, 0)),
            pl.BlockSpec((1, _NUM_ATTR, _CHUNKS[2], 128), lambda b: (b, 0, 0, 0)),
            pl.BlockSpec((1, _ROWS, 128), lambda b: (b, 0, 0)),
            pl.BlockSpec((1, 4, _ROWS, 128), lambda b: (b, 0, 0, 0)),
        ],
        out_specs=[
            pl.BlockSpec((1, _ROWS, 128), lambda b: (b, 0, 0)),
            pl.BlockSpec((8, 128), lambda b: (0, 0)),
        ],
        out_shape=[
            jax.ShapeDtypeStruct((_BATCH, _ROWS, 128), jnp.float32),
            jax.ShapeDtypeStruct((8, 128), jnp.float32),
        ],
    )(f0, f1, f2, conf, loc_t4)


def _sc_topk(lossneg, partials):
    mesh = plsc.VectorSubcoreMesh(core_axis_name="c", subcore_axis_name="s")

    @functools.partial(
        pl.kernel,
        mesh=mesh,
        out_type=jax.ShapeDtypeStruct((_BATCH, _L), jnp.float32),
        scratch_types=[
            pltpu.VMEM((_NUM_PRIORS,), jnp.float32),
            pltpu.VMEM((8, 128), jnp.float32),
            pltpu.VMEM((_L,), jnp.float32),
        ],
        compiler_params=pltpu.CompilerParams(needs_layout_passes=False),
    )
    def body(loss_hbm, part_hbm, out_hbm, loss_v, part_v, res_v):
        cid = lax.axis_index("c")
        sid = lax.axis_index("s")
        w = sid * 2 + cid
        pltpu.sync_copy(loss_hbm.at[w], loss_v)
        pltpu.sync_copy(part_hbm, part_v)

        lane = lax.iota(jnp.int32, 16)
        w_div = w // 4
        r1 = (w % 4) * 4 + 1
        r2 = (w % 4) * 4 + 2
        cntvec = jnp.zeros((_L,), jnp.float32)
        ownvec = jnp.zeros((_L,), jnp.float32)
        for j in range(8):
            v = part_v[0, j * _L:(j + 1) * _L]
            cntvec = cntvec + jnp.where(lane % 4 == 0, v, 0.0)
            ownvec = ownvec + jnp.where(
                (w_div == j) & ((lane == r1) | (lane == r2)), v, 0.0)
        num_pos_f = jnp.sum(cntvec)
        own = jnp.sum(ownvec)
        num_pos = num_pos_f.astype(jnp.int32)
        num_neg = jnp.minimum(_NEGPOS_RATIO * num_pos, _NUM_PRIORS - num_pos)

        def count_ge(cand):
            def inner(j, cacc):
                v = loss_v[pl.ds(j * _L, _L)]
                b = lax.bitcast_convert_type(v, jnp.int32)
                return cacc + jnp.where(b >= cand, 1, 0)
            cvec = lax.fori_loop(0, _NV, inner, jnp.zeros((_L,), jnp.int32))
            return jnp.sum(cvec)

        def bit_step(i, t):
            cand = t | (jnp.int32(1) << (jnp.int32(30) - i))
            cnt = count_ge(cand)
            return jnp.where(cnt >= num_neg, cand, t)

        t_key = lax.fori_loop(0, 31, bit_step, jnp.int32(0))

        def final(j, carry):
            sacc, cacc = carry
            v = loss_v[pl.ds(j * _L, _L)]
            b = lax.bitcast_convert_type(v, jnp.int32)
            gt = b > t_key
            return (sacc + jnp.where(gt, v, 0.0),
                    cacc + jnp.where(gt, 1, 0))

        sacc, cacc = lax.fori_loop(
            0, _NV, final,
            (jnp.zeros((_L,), jnp.float32), jnp.zeros((_L,), jnp.int32)))
        sum_gt = jnp.sum(sacc)
        cnt_gt = jnp.sum(cacc)
        t_vec = lax.bitcast_convert_type(
            jnp.full((_L,), t_key, jnp.int32), jnp.float32)
        t_val = jnp.sum(jnp.where(lane == 0, t_vec, 0.0))
        neg_sum = sum_gt + (num_neg - cnt_gt).astype(jnp.float32) * t_val
        neg_sum = jnp.where(num_neg > 0, neg_sum, 0.0)

        total_w = own + neg_sum
        res_v[...] = jnp.where(lane == 0, total_w,
                               jnp.where(lane == 1, num_pos_f, 0.0))
        pltpu.sync_copy(res_v, out_hbm.at[w])

    return body(lossneg, partials)


def kernel(feat0, feat1, feat2, loc_t, conf_t):
    b = feat0.shape[0]
    f0 = feat0.reshape(b, _NUM_ATTR, _CHUNKS[0], 128)
    f1 = feat1.reshape(b, _NUM_ATTR, _CHUNKS[1], 128)
    f2 = feat2.reshape(b, _NUM_ATTR, _CHUNKS[2], 128)
    conf = conf_t.reshape(b, _ROWS, 128).astype(jnp.int32)
    loc_t4 = jnp.transpose(loc_t, (0, 2, 1)).reshape(b, 4, _ROWS, 128)

    lossneg, partials = _dense(f0, f1, f2, conf, loc_t4)
    out = _sc_topk(lossneg.reshape(b, _NUM_PRIORS), partials)
    num_pos_f = out[0, 1]
    denom = jnp.maximum(num_pos_f, 1.0)
    return jnp.sum(out[:, 0]) / denom


# speedup vs baseline: 4.0750x; 1.2180x over previous
import functools

import jax
import jax.numpy as jnp
from jax import lax
from jax.experimental import pallas as pl
from jax.experimental.pallas import tpu as pltpu
from jax.experimental.pallas import tpu_sc as plsc

_NUM_CLASSES = 10
_NUM_ATTR = _NUM_CLASSES + 4
_NUM_PRIORS = 64 * 64 + 32 * 32 + 16 * 16
_NEGPOS_RATIO = 3
_BATCH = 32
_L = 16
_NV = _NUM_PRIORS // _L
_ROWS = _NUM_PRIORS // 128
_CHUNKS = (32, 8, 2)


def _dense_body(f0_ref, f1_ref, f2_ref, conf_ref, loc_ref,
                lossneg_ref, part_ref):
    b = pl.program_id(0)
    c_all = conf_ref[0]
    l_all = loc_ref[0]

    pos_cnt = jnp.float32(0.0)
    pos_sum = jnp.float32(0.0)
    loc_sum = jnp.float32(0.0)
    off = 0
    for x_ref, h in zip((f0_ref, f1_ref, f2_ref), _CHUNKS):
        x = x_ref[0]
        cc = c_all[off:off + h]
        lc = l_all[:, off:off + h, :]
        conf = x[4:]
        m = jnp.max(conf, axis=0)
        s = jnp.sum(jnp.exp(conf - m[None]), axis=0)
        lse = m + jnp.log(s)
        katt = lax.broadcasted_iota(jnp.int32, (_NUM_CLASSES, h, 128), 0)
        gathered = jnp.max(
            jnp.where(katt == cc[None], conf, -jnp.inf), axis=0)
        loss_c_all = lse - gathered
        pos = cc > 0
        posf = pos.astype(jnp.float32)
        lossneg_ref[0, off:off + h, :] = jnp.where(pos, 0.0, loss_c_all)
        pos_cnt += jnp.sum(posf)
        pos_sum += jnp.sum(loss_c_all * posf)
        d = x[:4] - lc
        ad = jnp.abs(d)
        sl1 = jnp.where(ad < 1.0, 0.5 * d * d, ad - 0.5)
        loc_sum += jnp.sum(sl1 * posf[None])
        off += h

    sub = lax.broadcasted_iota(jnp.int32, (8, 128), 0)
    lane = lax.broadcasted_iota(jnp.int32, (8, 128), 1)
    vals = jnp.where(lane == 4 * b, pos_cnt,
                     jnp.where(lane == 4 * b + 1, pos_sum,
                               jnp.where(lane == 4 * b + 2, loc_sum, 0.0)))
    vals = jnp.where(sub == 0, vals, 0.0)
    mask = (sub == 0) & (lane >= 4 * b) & (lane <= 4 * b + 2)

    @pl.when(b == 0)
    def _():
        part_ref[...] = vals

    @pl.when(b > 0)
    def _():
        part_ref[...] = jnp.where(mask, vals, part_ref[...])


def _dense(f0, f1, f2, conf, loc_t4):
    return pl.pallas_call(
        _dense_body,
        grid=(_BATCH,),
        in_specs=[
            pl.BlockSpec((1, _NUM_ATTR, _CHUNKS[0], 128), lambda b: (b, 0, 0, 0)),
            pl.BlockSpec((1, _NUM_ATTR, _CHUNKS[1], 128), lambda b: (b, 0, 0, 0)),
            pl.BlockSpec((1, _NUM_ATTR, _CHUNKS[2], 128), lambda b: (b, 0, 0, 0)),
            pl.BlockSpec((1, _ROWS, 128), lambda b: (b, 0, 0)),
            pl.BlockSpec((1, 4, _ROWS, 128), lambda b: (b, 0, 0, 0)),
        ],
        out_specs=[
            pl.BlockSpec((1, _ROWS, 128), lambda b: (b, 0, 0)),
            pl.BlockSpec((8, 128), lambda b: (0, 0)),
        ],
        out_shape=[
            jax.ShapeDtypeStruct((_BATCH, _ROWS, 128), jnp.float32),
            jax.ShapeDtypeStruct((8, 128), jnp.float32),
        ],
    )(f0, f1, f2, conf, loc_t4)


def _sc_topk(lossneg, partials):
    mesh = plsc.VectorSubcoreMesh(core_axis_name="c", subcore_axis_name="s")

    @functools.partial(
        pl.kernel,
        mesh=mesh,
        out_type=jax.ShapeDtypeStruct((_BATCH, _L), jnp.float32),
        scratch_types=[
            pltpu.VMEM((_NUM_PRIORS,), jnp.float32),
            pltpu.VMEM((8, 128), jnp.float32),
            pltpu.VMEM((_L,), jnp.float32),
            pltpu.VMEM((256,), jnp.int32),
            pltpu.VMEM((256,), jnp.float32),
        ],
        compiler_params=pltpu.CompilerParams(needs_layout_passes=False),
    )
    def body(loss_hbm, part_hbm, out_hbm, loss_v, part_v, res_v, hc, hs):
        cid = lax.axis_index("c")
        sid = lax.axis_index("s")
        w = sid * 2 + cid
        pltpu.sync_copy(loss_hbm.at[w], loss_v)
        pltpu.sync_copy(part_hbm, part_v)

        lane = lax.iota(jnp.int32, 16)
        w_div = w // 4
        r1 = (w % 4) * 4 + 1
        r2 = (w % 4) * 4 + 2
        cntvec = jnp.zeros((_L,), jnp.float32)
        ownvec = jnp.zeros((_L,), jnp.float32)
        for j in range(8):
            v = part_v[0, j * _L:(j + 1) * _L]
            cntvec = cntvec + jnp.where(lane % 4 == 0, v, 0.0)
            ownvec = ownvec + jnp.where(
                (w_div == j) & ((lane == r1) | (lane == r2)), v, 0.0)
        num_pos_f = jnp.sum(cntvec)
        own = jnp.sum(ownvec)
        num_pos = num_pos_f.astype(jnp.int32)
        num_neg = jnp.minimum(_NEGPOS_RATIO * num_pos, _NUM_PRIORS - num_pos)

        ones = jnp.full((_L,), 1, jnp.int32)
        zc = jnp.zeros((_L,), jnp.int32)
        zs = jnp.zeros((_L,), jnp.float32)
        prefix = jnp.int32(0)
        cnt_gt = jnp.int32(0)
        sum_gt = jnp.float32(0.0)
        k_lvl = num_neg
        for lvl, (sh, wbits) in enumerate(((23, 8), (15, 8), (7, 8), (0, 7))):
            for i in range(16):
                hc[pl.ds(i * _L, _L)] = zc
                hs[pl.ds(i * _L, _L)] = zs

            bmask = jnp.int32((1 << wbits) - 1)

            def scan(j, carry, sh=sh, lvl=lvl, bmask=bmask):
                pfx = carry
                v = loss_v[pl.ds(j * _L, _L)]
                b = lax.bitcast_convert_type(v, jnp.int32)
                bk = (b >> sh) & bmask
                if lvl == 0:
                    plsc.addupdate_scatter(hc, [bk], ones)
                    plsc.addupdate_scatter(hs, [bk], v)
                else:
                    m = (b >> (sh + wbits)) == pfx
                    plsc.addupdate_scatter(hc, [bk], ones, mask=m)
                    plsc.addupdate_scatter(hs, [bk], v, mask=m)
                return carry

            lax.fori_loop(0, _NV, scan, prefix)

            s0c = jnp.int32(0)
            s0s = jnp.float32(0.0)
            bsel_v = zc
            c_ab_v = zc
            s_ab_v = zs
            for i in range(15, -1, -1):
                cblk = hc[pl.ds(i * _L, _L)]
                sblk = hs[pl.ds(i * _L, _L)]
                rc = jnp.flip(cblk)
                rs = jnp.flip(sblk)
                cumc = jnp.cumsum(rc)
                cums = jnp.cumsum(rs)
                prev_c = s0c + cumc - rc
                prev_s = s0s + cums - rs
                crossed = (prev_c < k_lvl) & (prev_c + rc >= k_lvl)
                bucket_id = jnp.int32(i * _L + 15) - lane
                bsel_v = bsel_v + jnp.where(crossed, bucket_id, 0)
                c_ab_v = c_ab_v + jnp.where(crossed, prev_c, 0)
                s_ab_v = s_ab_v + jnp.where(crossed, prev_s, 0.0)
                s0c = s0c + jnp.sum(cblk)
                s0s = s0s + jnp.sum(sblk)
            bsel = jnp.sum(bsel_v)
            c_ab = jnp.sum(c_ab_v)
            s_ab = jnp.sum(s_ab_v)
            prefix = (prefix << wbits) | bsel
            cnt_gt = cnt_gt + c_ab
            sum_gt = sum_gt + s_ab
            k_lvl = k_lvl - c_ab

        t_vec = lax.bitcast_convert_type(
            jnp.full((_L,), prefix, jnp.int32), jnp.float32)
        t_val = jnp.sum(jnp.where(lane == 0, t_vec, 0.0))
        neg_sum = sum_gt + (num_neg - cnt_gt).astype(jnp.float32) * t_val
        neg_sum = jnp.where(num_neg > 0, neg_sum, 0.0)

        total_w = own + neg_sum
        res_v[...] = jnp.where(lane == 0, total_w,
                               jnp.where(lane == 1, num_pos_f, 0.0))
        pltpu.sync_copy(res_v, out_hbm.at[w])

    return body(lossneg, partials)


def kernel(feat0, feat1, feat2, loc_t, conf_t):
    b = feat0.shape[0]
    f0 = feat0.reshape(b, _NUM_ATTR, _CHUNKS[0], 128)
    f1 = feat1.reshape(b, _NUM_ATTR, _CHUNKS[1], 128)
    f2 = feat2.reshape(b, _NUM_ATTR, _CHUNKS[2], 128)
    conf = conf_t.reshape(b, _ROWS, 128).astype(jnp.int32)
    loc_t4 = jnp.transpose(loc_t, (0, 2, 1)).reshape(b, 4, _ROWS, 128)

    lossneg, partials = _dense(f0, f1, f2, conf, loc_t4)
    out = _sc_topk(lossneg.reshape(b, _NUM_PRIORS), partials)
    num_pos_f = out[0, 1]
    denom = jnp.maximum(num_pos_f, 1.0)
    return jnp.sum(out[:, 0]) / denom


# speedup vs baseline: 4.1081x; 1.0081x over previous
import functools

import jax
import jax.numpy as jnp
from jax import lax
from jax.experimental import pallas as pl
from jax.experimental.pallas import tpu as pltpu
from jax.experimental.pallas import tpu_sc as plsc

_NUM_CLASSES = 10
_NUM_ATTR = _NUM_CLASSES + 4
_NUM_PRIORS = 64 * 64 + 32 * 32 + 16 * 16
_NEGPOS_RATIO = 3
_BATCH = 32
_L = 16
_HW = ((64, 64), (32, 32), (16, 16))


def _dense_body(f0_ref, f1_ref, f2_ref, c0_ref, c1_ref, c2_ref,
                l0_ref, l1_ref, l2_ref,
                o0_ref, o1_ref, o2_ref, part_ref):
    b = pl.program_id(0)

    pos_cnt = jnp.float32(0.0)
    pos_sum = jnp.float32(0.0)
    loc_sum = jnp.float32(0.0)
    for x_ref, c_ref, l_ref, o_ref, (hh, ww) in zip(
            (f0_ref, f1_ref, f2_ref), (c0_ref, c1_ref, c2_ref),
            (l0_ref, l1_ref, l2_ref), (o0_ref, o1_ref, o2_ref), _HW):
        x = x_ref[0]
        cc = c_ref[0]
        lc = l_ref[0]
        conf = x[4:]
        m = jnp.max(conf, axis=0)
        s = jnp.sum(jnp.exp(conf - m[None]), axis=0)
        lse = m + jnp.log(s)
        katt = lax.broadcasted_iota(jnp.int32, (_NUM_CLASSES, hh, ww), 0)
        gathered = jnp.max(
            jnp.where(katt == cc[None], conf, -jnp.inf), axis=0)
        loss_c_all = lse - gathered
        pos = cc > 0
        posf = pos.astype(jnp.float32)
        o_ref[0] = jnp.where(pos, 0.0, loss_c_all)
        pos_cnt += jnp.sum(posf)
        pos_sum += jnp.sum(loss_c_all * posf)
        d = x[:4] - lc
        ad = jnp.abs(d)
        sl1 = jnp.where(ad < 1.0, 0.5 * d * d, ad - 0.5)
        loc_sum += jnp.sum(sl1 * posf[None])

    sub = lax.broadcasted_iota(jnp.int32, (8, 128), 0)
    lane = lax.broadcasted_iota(jnp.int32, (8, 128), 1)
    vals = jnp.where(lane == 4 * b, pos_cnt,
                     jnp.where(lane == 4 * b + 1, pos_sum,
                               jnp.where(lane == 4 * b + 2, loc_sum, 0.0)))
    vals = jnp.where(sub == 0, vals, 0.0)
    mask = (sub == 0) & (lane >= 4 * b) & (lane <= 4 * b + 2)

    @pl.when(b == 0)
    def _():
        part_ref[...] = vals

    @pl.when(b > 0)
    def _():
        part_ref[...] = jnp.where(mask, vals, part_ref[...])


def _dense(f0, f1, f2, c0, c1, c2, l0, l1, l2):
    specs_f = [pl.BlockSpec((1, _NUM_ATTR, h, w), lambda b: (b, 0, 0, 0))
               for h, w in _HW]
    specs_c = [pl.BlockSpec((1, h, w), lambda b: (b, 0, 0)) for h, w in _HW]
    specs_l = [pl.BlockSpec((1, 4, h, w), lambda b: (b, 0, 0, 0))
               for h, w in _HW]
    return pl.pallas_call(
        _dense_body,
        grid=(_BATCH,),
        in_specs=specs_f + specs_c + specs_l,
        out_specs=specs_c + [pl.BlockSpec((8, 128), lambda b: (0, 0))],
        out_shape=[jax.ShapeDtypeStruct((_BATCH, h, w), jnp.float32)
                   for h, w in _HW]
        + [jax.ShapeDtypeStruct((8, 128), jnp.float32)],
    )(f0, f1, f2, c0, c1, c2, l0, l1, l2)


def _sc_topk(loss0, loss1, loss2, partials):
    mesh = plsc.VectorSubcoreMesh(core_axis_name="c", subcore_axis_name="s")

    @functools.partial(
        pl.kernel,
        mesh=mesh,
        out_type=jax.ShapeDtypeStruct((_BATCH, _L), jnp.float32),
        scratch_types=[
            pltpu.VMEM((64, 64), jnp.float32),
            pltpu.VMEM((32, 32), jnp.float32),
            pltpu.VMEM((16, 16), jnp.float32),
            pltpu.VMEM((8, 128), jnp.float32),
            pltpu.VMEM((_L,), jnp.float32),
            pltpu.VMEM((256,), jnp.int32),
            pltpu.VMEM((256,), jnp.float32),
        ],
        compiler_params=pltpu.CompilerParams(needs_layout_passes=False),
    )
    def body(l0_hbm, l1_hbm, l2_hbm, part_hbm, out_hbm,
             v0, v1, v2, part_v, res_v, hc, hs):
        cid = lax.axis_index("c")
        sid = lax.axis_index("s")
        w = sid * 2 + cid
        pltpu.sync_copy(l0_hbm.at[w], v0)
        pltpu.sync_copy(l1_hbm.at[w], v1)
        pltpu.sync_copy(l2_hbm.at[w], v2)
        pltpu.sync_copy(part_hbm, part_v)

        lane = lax.iota(jnp.int32, 16)
        w_div = w // 4
        r1 = (w % 4) * 4 + 1
        r2 = (w % 4) * 4 + 2
        cntvec = jnp.zeros((_L,), jnp.float32)
        ownvec = jnp.zeros((_L,), jnp.float32)
        for j in range(8):
            v = part_v[0, j * _L:(j + 1) * _L]
            cntvec = cntvec + jnp.where(lane % 4 == 0, v, 0.0)
            ownvec = ownvec + jnp.where(
                (w_div == j) & ((lane == r1) | (lane == r2)), v, 0.0)
        num_pos_f = jnp.sum(cntvec)
        own = jnp.sum(ownvec)
        num_pos = num_pos_f.astype(jnp.int32)
        num_neg = jnp.minimum(_NEGPOS_RATIO * num_pos, _NUM_PRIORS - num_pos)

        ones = jnp.full((_L,), 1, jnp.int32)
        zc = jnp.zeros((_L,), jnp.int32)
        zs = jnp.zeros((_L,), jnp.float32)
        prefix = jnp.int32(0)
        cnt_gt = jnp.int32(0)
        sum_gt = jnp.float32(0.0)
        k_lvl = num_neg
        for lvl, (sh, wbits) in enumerate(((23, 8), (15, 8), (7, 8), (0, 7))):
            for i in range(16):
                hc[pl.ds(i * _L, _L)] = zc
                hs[pl.ds(i * _L, _L)] = zs

            bmask = jnp.int32((1 << wbits) - 1)

            def scan(j, carry, ref=None, ncol=0, sh=sh, lvl=lvl, bmask=bmask,
                     wbits=wbits):
                pfx = carry
                r = j // ncol
                c = (j % ncol) * _L
                v = ref[r, pl.ds(c, _L)]
                b = lax.bitcast_convert_type(v, jnp.int32)
                bk = (b >> sh) & bmask
                if lvl == 0:
                    plsc.addupdate_scatter(hc, [bk], ones)
                    plsc.addupdate_scatter(hs, [bk], v)
                else:
                    m = (b >> (sh + wbits)) == pfx
                    plsc.addupdate_scatter(hc, [bk], ones, mask=m)
                    plsc.addupdate_scatter(hs, [bk], v, mask=m)
                return carry

            for ref, (hh, ww) in zip((v0, v1, v2), _HW):
                lax.fori_loop(
                    0, hh * ww // _L,
                    functools.partial(scan, ref=ref, ncol=ww // _L), prefix)

            s0c = jnp.int32(0)
            s0s = jnp.float32(0.0)
            bsel_v = zc
            c_ab_v = zc
            s_ab_v = zs
            for i in range(15, -1, -1):
                cblk = hc[pl.ds(i * _L, _L)]
                sblk = hs[pl.ds(i * _L, _L)]
                rc = jnp.flip(cblk)
                rs = jnp.flip(sblk)
                cumc = jnp.cumsum(rc)
                cums = jnp.cumsum(rs)
                prev_c = s0c + cumc - rc
                prev_s = s0s + cums - rs
                crossed = (prev_c < k_lvl) & (prev_c + rc >= k_lvl)
                bucket_id = jnp.int32(i * _L + 15) - lane
                bsel_v = bsel_v + jnp.where(crossed, bucket_id, 0)
                c_ab_v = c_ab_v + jnp.where(crossed, prev_c, 0)
                s_ab_v = s_ab_v + jnp.where(crossed, prev_s, 0.0)
                s0c = s0c + jnp.sum(cblk)
                s0s = s0s + jnp.sum(sblk)
            bsel = jnp.sum(bsel_v)
            c_ab = jnp.sum(c_ab_v)
            s_ab = jnp.sum(s_ab_v)
            prefix = (prefix << wbits) | bsel
            cnt_gt = cnt_gt + c_ab
            sum_gt = sum_gt + s_ab
            k_lvl = k_lvl - c_ab

        t_vec = lax.bitcast_convert_type(
            jnp.full((_L,), prefix, jnp.int32), jnp.float32)
        t_val = jnp.sum(jnp.where(lane == 0, t_vec, 0.0))
        neg_sum = sum_gt + (num_neg - cnt_gt).astype(jnp.float32) * t_val
        neg_sum = jnp.where(num_neg > 0, neg_sum, 0.0)

        total_w = own + neg_sum
        res_v[...] = jnp.where(lane == 0, total_w,
                               jnp.where(lane == 1, num_pos_f, 0.0))
        pltpu.sync_copy(res_v, out_hbm.at[w])

    return body(loss0, loss1, loss2, partials)


def kernel(feat0, feat1, feat2, loc_t, conf_t):
    b = feat0.shape[0]
    ci = conf_t.astype(jnp.int32)
    sizes = (4096, 1024, 256)
    offs = (0, 4096, 5120)
    confs = [ci[:, o:o + n].reshape(b, h, w)
             for (o, n, (h, w)) in zip(offs, sizes, _HW)]
    locs = [loc_t[:, o:o + n, :].transpose(0, 2, 1).reshape(b, 4, h, w)
            for (o, n, (h, w)) in zip(offs, sizes, _HW)]

    loss0, loss1, loss2, partials = _dense(feat0, feat1, feat2,
                                           *confs, *locs)
    out = _sc_topk(loss0, loss1, loss2, partials)
    num_pos_f = out[0, 1]
    denom = jnp.maximum(num_pos_f, 1.0)
    return jnp.sum(out[:, 0]) / denom
